# baseline (speedup 1.0000x reference)
import dataclasses
import functools

import jax
import jax.numpy as jnp
from jax import lax
from jax.experimental import pallas as pl
from jax.experimental.pallas import tpu as pltpu
from jax.experimental.pallas import tpu_sc as plsc

_N = 10000
_L = 128
_NC = 2
_NS = 16
_CHUNK = 80
_NBUF = 2

_NTAB = 10112
_ROWS_PER_TEC = _NTAB // _NS
_DUMMY = _N


def _ceil_to(x, m):
    return (x + m - 1) // m * m


def _tc_pre(x, h0, w0, Wh, W1, W2, we, W3):
    n = x.shape[0]
    blk = 2000
    grid = n // blk

    def body(x_ref, h_ref, w0_ref, Wh_ref, W1_ref, W2_ref, we_ref, W3_ref,
             ne_ref, a_ref, b_ref, vv_ref):
        ne = jnp.maximum(
            x_ref[...] * w0_ref[...]
            + jnp.dot(h_ref[...], Wh_ref[...], preferred_element_type=jnp.float32),
            0.0)
        ne_ref[...] = ne
        a_ref[...] = jnp.dot(ne, W1_ref[...], preferred_element_type=jnp.float32)
        b_ref[...] = jnp.dot(ne, W2_ref[...], preferred_element_type=jnp.float32)
        wev = we_ref[...]
        wcat = jnp.concatenate([jnp.maximum(wev, 0.0), jnp.minimum(wev, 0.0)], axis=0)
        vv_ref[...] = jnp.dot(wcat, W3_ref[...], preferred_element_type=jnp.float32)

    f32 = jnp.float32
    return pl.pallas_call(
        body,
        grid=(grid,),
        in_specs=[
            pl.BlockSpec((blk, 1), lambda i: (i, 0)),
            pl.BlockSpec((blk, _L), lambda i: (i, 0)),
            pl.BlockSpec((1, _L), lambda i: (0, 0)),
            pl.BlockSpec((_L, _L), lambda i: (0, 0)),
            pl.BlockSpec((_L, _L), lambda i: (0, 0)),
            pl.BlockSpec((_L, _L), lambda i: (0, 0)),
            pl.BlockSpec((1, _L), lambda i: (0, 0)),
            pl.BlockSpec((_L, _L), lambda i: (0, 0)),
        ],
        out_specs=[
            pl.BlockSpec((blk, _L), lambda i: (i, 0)),
            pl.BlockSpec((blk, _L), lambda i: (i, 0)),
            pl.BlockSpec((blk, _L), lambda i: (i, 0)),
            pl.BlockSpec((2, _L), lambda i: (0, 0)),
        ],
        out_shape=[
            jax.ShapeDtypeStruct((n, _L), f32),
            jax.ShapeDtypeStruct((n, _L), f32),
            jax.ShapeDtypeStruct((n, _L), f32),
            jax.ShapeDtypeStruct((2, _L), f32),
        ],
    )(x, h0, w0, Wh, W1, W2, we, W3)


def _sc_edge(A, B, vv_flat, src, dst, ev, chunks_per_tec):
    epad = src.shape[0]
    e_core = epad // _NC
    e_tec = e_core // _NS
    mesh = plsc.VectorSubcoreMesh(core_axis_name="c", subcore_axis_name="s")
    cp = pltpu.CompilerParams()
    if "needs_layout_passes" in pltpu.CompilerParams.__dataclass_fields__:
        cp = dataclasses.replace(cp, needs_layout_passes=False)

    @functools.partial(
        pl.kernel,
        out_type=jax.ShapeDtypeStruct((_NC, _NTAB, _L), jnp.float32),
        mesh=mesh,
        compiler_params=cp,
        scratch_types=[
            pltpu.VMEM_SHARED((_NTAB, _L), jnp.float32),
            pltpu.VMEM((_NBUF, _CHUNK), jnp.int32),
            pltpu.VMEM((_NBUF, _CHUNK), jnp.int32),
            pltpu.VMEM((_NBUF, _CHUNK), jnp.float32),
            pltpu.VMEM((_NBUF, _CHUNK, _L), jnp.float32),
            pltpu.VMEM((_NBUF, _CHUNK, _L), jnp.float32),
            pltpu.VMEM((2 * _L,), jnp.float32),
            pltpu.SemaphoreType.DMA((_NBUF,)),
            pltpu.SemaphoreType.DMA((_NBUF,)),
            pltpu.SemaphoreType.DMA,
        ],
    )
    def sck(a_hbm, b_hbm, vv_hbm, src_hbm, dst_hbm, e_hbm, out_hbm,
            agg_sh, srcb, dstb, eb, ab, bb, vvb, isem, gsem, msem):
        c = lax.axis_index("c")
        sid = lax.axis_index("s")
        base = c * e_core + sid * e_tec

        pltpu.async_copy(vv_hbm, vvb, msem).wait()
        vp = [vvb[pl.ds(16 * j, 16)] for j in range(8)]
        vn = [vvb[pl.ds(_L + 16 * j, 16)] for j in range(8)]

        zero = jnp.zeros((16,), jnp.float32)

        @pl.loop(0, _CHUNK)
        def _(r):
            for j in range(8):
                ab[0, r, pl.ds(16 * j, 16)] = zero

        row0 = sid * _ROWS_PER_TEC
        for i in range(_ROWS_PER_TEC // _CHUNK):
            pltpu.sync_copy(ab.at[0], agg_sh.at[pl.ds(row0 + i * _CHUNK, _CHUNK)])
        rem = _ROWS_PER_TEC % _CHUNK
        if rem:
            full = (_ROWS_PER_TEC // _CHUNK) * _CHUNK
            pltpu.sync_copy(ab.at[0, pl.ds(0, rem)],
                            agg_sh.at[pl.ds(row0 + full, rem)])
        plsc.subcore_barrier()

        def issue_idx(t, q):
            off = base + t * _CHUNK
            pltpu.async_copy(src_hbm.at[pl.ds(off, _CHUNK)], srcb.at[q], isem.at[q])
            pltpu.async_copy(dst_hbm.at[pl.ds(off, _CHUNK)], dstb.at[q], isem.at[q])
            pltpu.async_copy(e_hbm.at[pl.ds(off, _CHUNK)], eb.at[q], isem.at[q])

        def wait_idx(t, q):
            off = base + t * _CHUNK
            pltpu.make_async_copy(src_hbm.at[pl.ds(off, _CHUNK)], srcb.at[q], isem.at[q]).wait()
            pltpu.make_async_copy(dst_hbm.at[pl.ds(off, _CHUNK)], dstb.at[q], isem.at[q]).wait()
            pltpu.make_async_copy(e_hbm.at[pl.ds(off, _CHUNK)], eb.at[q], isem.at[q]).wait()

        def issue_gather(q):
            pltpu.async_copy(a_hbm.at[srcb.at[q]], ab.at[q], gsem.at[q])
            pltpu.async_copy(b_hbm.at[dstb.at[q]], bb.at[q], gsem.at[q])

        def wait_gather(q):
            pltpu.make_async_copy(a_hbm.at[srcb.at[q]], ab.at[q], gsem.at[q]).wait()
            pltpu.make_async_copy(b_hbm.at[dstb.at[q]], bb.at[q], gsem.at[q]).wait()

        def compute(q):
            @plsc.parallel_loop(0, _CHUNK, step=1, unroll=8)
            def _(k):
                evec = plsc.load_gather(eb.at[q], [jnp.full((16,), k, jnp.int32)])
                sp = jnp.maximum(evec, 0.0)
                sn = jnp.minimum(evec, 0.0)
                for j in range(8):
                    s = pl.ds(16 * j, 16)
                    r = ab[q, k, s] + bb[q, k, s] + sp * vp[j] + sn * vn[j]
                    ab[q, k, s] = jnp.maximum(r, 0.0)

        def scatter(q):
            pltpu.sync_copy(ab.at[q], agg_sh.at[dstb.at[q]], add=True)

        issue_idx(0, 0)
        issue_idx(1, 1)
        wait_idx(0, 0)
        issue_gather(0)

        @pl.loop(0, chunks_per_tec // _NBUF)
        def _(g):
            for qq in range(_NBUF):
                t = g * _NBUF + qq
                nq = 1 - qq

                @pl.when(t + 1 < chunks_per_tec)
                def _():
                    wait_idx(t + 1, nq)
                    issue_gather(nq)

                wait_gather(qq)
                compute(qq)
                scatter(qq)

                @pl.when(t + 2 < chunks_per_tec)
                def _():
                    issue_idx(t + 2, qq)

        plsc.subcore_barrier()
        pltpu.sync_copy(agg_sh.at[pl.ds(row0, _ROWS_PER_TEC)],
                        out_hbm.at[c, pl.ds(row0, _ROWS_PER_TEC)])

    return sck(A, B, vv_flat, src, dst, ev)


def _tc_post(parts, ne, Wun, Wua, Wdn, Wdl, Wd2):
    n = ne.shape[0]
    blk = 2000
    grid = n // blk

    def body(p_ref, ne_ref, Wun_ref, Wua_ref, Wdn_ref, Wdl_ref, Wd2_ref, o_ref):
        agg = p_ref[0] + p_ref[1]
        ne = ne_ref[...]
        latent = jnp.maximum(
            jnp.dot(ne, Wun_ref[...], preferred_element_type=jnp.float32)
            + jnp.dot(agg, Wua_ref[...], preferred_element_type=jnp.float32), 0.0)
        dech = jnp.maximum(
            jnp.dot(ne, Wdn_ref[...], preferred_element_type=jnp.float32)
            + jnp.dot(latent, Wdl_ref[...], preferred_element_type=jnp.float32), 0.0)
        o_ref[...] = jnp.dot(dech, Wd2_ref[...], preferred_element_type=jnp.float32)

    return pl.pallas_call(
        body,
        grid=(grid,),
        in_specs=[
            pl.BlockSpec((2, blk, _L), lambda i: (0, i, 0)),
            pl.BlockSpec((blk, _L), lambda i: (i, 0)),
            pl.BlockSpec((_L, _L), lambda i: (0, 0)),
            pl.BlockSpec((_L, _L), lambda i: (0, 0)),
            pl.BlockSpec((_L, _L), lambda i: (0, 0)),
            pl.BlockSpec((_L, _L), lambda i: (0, 0)),
            pl.BlockSpec((_L, 1), lambda i: (0, 0)),
        ],
        out_specs=[pl.BlockSpec((blk, 1), lambda i: (i, 0))],
        out_shape=[jax.ShapeDtypeStruct((n, 1), jnp.float32)],
    )(parts, ne, Wun, Wua, Wdn, Wdl, Wd2)[0]


def kernel(node_features, edge_features, latent_features, edge_index,
           W_node_enc, W_edge_enc, W_msg, W_upd, W_dec1, W_dec2):
    n = node_features.shape[0]
    e_cnt = edge_features.shape[0]

    x = node_features.astype(jnp.float32)[:, None]
    h0 = latent_features.astype(jnp.float32)
    ev = edge_features.astype(jnp.float32)
    src = edge_index[0].astype(jnp.int32)
    dst = edge_index[1].astype(jnp.int32)

    epad = _ceil_to(e_cnt, _NC * _NS * _CHUNK * _NBUF)
    chunks_per_tec = epad // (_NC * _NS * _CHUNK)
    npad = epad - e_cnt
    src = jnp.concatenate([src, jnp.zeros((npad,), jnp.int32)])
    dst = jnp.concatenate([dst, jnp.full((npad,), _DUMMY, jnp.int32)])
    ev = jnp.concatenate([ev, jnp.zeros((npad,), jnp.float32)])

    w0 = W_node_enc[0:1]
    Wh = W_node_enc[1:]
    W1 = W_msg[0:_L]
    W2 = W_msg[_L:2 * _L]
    W3 = W_msg[2 * _L:]

    ne, A, B, vv = _tc_pre(x, h0, w0, Wh, W1, W2, W_edge_enc, W3)
    parts = _sc_edge(A, B, vv.reshape(-1), src, dst, ev, chunks_per_tec)
    out = _tc_post(parts, ne, W_upd[:_L], W_upd[_L:], W_dec1[:_L], W_dec1[_L:], W_dec2)
    return out


# speedup vs baseline: 1.0419x; 1.0419x over previous
import dataclasses
import functools

import jax
import jax.numpy as jnp
from jax import lax
from jax.experimental import pallas as pl
from jax.experimental.pallas import tpu as pltpu
from jax.experimental.pallas import tpu_sc as plsc

_N = 10000
_L = 128
_NC = 2
_NS = 16
_CHUNK = 56
_NBUF = 3

_NTAB = 10112
_ROWS_PER_TEC = _NTAB // _NS
_DUMMY = _N


def _ceil_to(x, m):
    return (x + m - 1) // m * m


def _tc_pre(x, h0, w0, Wh, W1, W2, we, W3):
    n = x.shape[0]
    blk = 2000
    grid = n // blk

    def body(x_ref, h_ref, w0_ref, Wh_ref, W1_ref, W2_ref, we_ref, W3_ref,
             ne_ref, a_ref, b_ref, vv_ref):
        ne = jnp.maximum(
            x_ref[...] * w0_ref[...]
            + jnp.dot(h_ref[...], Wh_ref[...], preferred_element_type=jnp.float32),
            0.0)
        ne_ref[...] = ne
        a_ref[...] = jnp.dot(ne, W1_ref[...], preferred_element_type=jnp.float32)
        b_ref[...] = jnp.dot(ne, W2_ref[...], preferred_element_type=jnp.float32)
        wev = we_ref[...]
        wcat = jnp.concatenate([jnp.maximum(wev, 0.0), jnp.minimum(wev, 0.0)], axis=0)
        vv_ref[...] = jnp.dot(wcat, W3_ref[...], preferred_element_type=jnp.float32)

    f32 = jnp.float32
    return pl.pallas_call(
        body,
        grid=(grid,),
        in_specs=[
            pl.BlockSpec((blk, 1), lambda i: (i, 0)),
            pl.BlockSpec((blk, _L), lambda i: (i, 0)),
            pl.BlockSpec((1, _L), lambda i: (0, 0)),
            pl.BlockSpec((_L, _L), lambda i: (0, 0)),
            pl.BlockSpec((_L, _L), lambda i: (0, 0)),
            pl.BlockSpec((_L, _L), lambda i: (0, 0)),
            pl.BlockSpec((1, _L), lambda i: (0, 0)),
            pl.BlockSpec((_L, _L), lambda i: (0, 0)),
        ],
        out_specs=[
            pl.BlockSpec((blk, _L), lambda i: (i, 0)),
            pl.BlockSpec((blk, _L), lambda i: (i, 0)),
            pl.BlockSpec((blk, _L), lambda i: (i, 0)),
            pl.BlockSpec((2, _L), lambda i: (0, 0)),
        ],
        out_shape=[
            jax.ShapeDtypeStruct((n, _L), f32),
            jax.ShapeDtypeStruct((n, _L), f32),
            jax.ShapeDtypeStruct((n, _L), f32),
            jax.ShapeDtypeStruct((2, _L), f32),
        ],
    )(x, h0, w0, Wh, W1, W2, we, W3)


def _sc_edge(A, B, vv_flat, src, dst, ev, chunks_per_tec):
    epad = src.shape[0]
    e_core = epad // _NC
    e_tec = e_core // _NS
    mesh = plsc.VectorSubcoreMesh(core_axis_name="c", subcore_axis_name="s")
    cp = pltpu.CompilerParams()
    if "needs_layout_passes" in pltpu.CompilerParams.__dataclass_fields__:
        cp = dataclasses.replace(cp, needs_layout_passes=False)

    @functools.partial(
        pl.kernel,
        out_type=jax.ShapeDtypeStruct((_NC, _NTAB, _L), jnp.float32),
        mesh=mesh,
        compiler_params=cp,
        scratch_types=[
            pltpu.VMEM_SHARED((_NTAB, _L), jnp.float32),
            pltpu.VMEM((_NBUF, _CHUNK), jnp.int32),
            pltpu.VMEM((_NBUF, _CHUNK), jnp.int32),
            pltpu.VMEM((_NBUF, _CHUNK), jnp.float32),
            pltpu.VMEM((_NBUF, _CHUNK, _L), jnp.float32),
            pltpu.VMEM((_NBUF, _CHUNK, _L), jnp.float32),
            pltpu.VMEM((2 * _L,), jnp.float32),
            pltpu.SemaphoreType.DMA((_NBUF,)),
            pltpu.SemaphoreType.DMA((_NBUF,)),
            pltpu.SemaphoreType.DMA((_NBUF,)),
            pltpu.SemaphoreType.DMA,
        ],
    )
    def sck(a_hbm, b_hbm, vv_hbm, src_hbm, dst_hbm, e_hbm, out_hbm,
            agg_sh, srcb, dstb, eb, ab, bb, vvb, isem, gsem, ssem, msem):
        c = lax.axis_index("c")
        sid = lax.axis_index("s")
        base = c * e_core + sid * e_tec

        pltpu.async_copy(vv_hbm, vvb, msem).wait()
        vp = [vvb[pl.ds(16 * j, 16)] for j in range(8)]
        vn = [vvb[pl.ds(_L + 16 * j, 16)] for j in range(8)]

        zero = jnp.zeros((16,), jnp.float32)

        @pl.loop(0, _CHUNK)
        def _(r):
            for j in range(8):
                ab[0, r, pl.ds(16 * j, 16)] = zero

        row0 = sid * _ROWS_PER_TEC
        for i in range(_ROWS_PER_TEC // _CHUNK):
            pltpu.sync_copy(ab.at[0], agg_sh.at[pl.ds(row0 + i * _CHUNK, _CHUNK)])
        rem = _ROWS_PER_TEC % _CHUNK
        if rem:
            full = (_ROWS_PER_TEC // _CHUNK) * _CHUNK
            pltpu.sync_copy(ab.at[0, pl.ds(0, rem)],
                            agg_sh.at[pl.ds(row0 + full, rem)])
        plsc.subcore_barrier()

        def issue_idx(t, q):
            off = base + t * _CHUNK
            pltpu.async_copy(src_hbm.at[pl.ds(off, _CHUNK)], srcb.at[q], isem.at[q])
            pltpu.async_copy(dst_hbm.at[pl.ds(off, _CHUNK)], dstb.at[q], isem.at[q])
            pltpu.async_copy(e_hbm.at[pl.ds(off, _CHUNK)], eb.at[q], isem.at[q])

        def wait_idx(t, q):
            off = base + t * _CHUNK
            pltpu.make_async_copy(src_hbm.at[pl.ds(off, _CHUNK)], srcb.at[q], isem.at[q]).wait()
            pltpu.make_async_copy(dst_hbm.at[pl.ds(off, _CHUNK)], dstb.at[q], isem.at[q]).wait()
            pltpu.make_async_copy(e_hbm.at[pl.ds(off, _CHUNK)], eb.at[q], isem.at[q]).wait()

        def issue_gather(q):
            pltpu.async_copy(a_hbm.at[srcb.at[q]], ab.at[q], gsem.at[q])
            pltpu.async_copy(b_hbm.at[dstb.at[q]], bb.at[q], gsem.at[q])

        def wait_gather(q):
            pltpu.make_async_copy(a_hbm.at[srcb.at[q]], ab.at[q], gsem.at[q]).wait()
            pltpu.make_async_copy(b_hbm.at[dstb.at[q]], bb.at[q], gsem.at[q]).wait()

        def compute(q):
            @plsc.parallel_loop(0, _CHUNK, step=1, unroll=4)
            def _(k):
                evec = plsc.load_gather(eb.at[q], [jnp.full((16,), k, jnp.int32)])
                sp = jnp.maximum(evec, 0.0)
                sn = jnp.minimum(evec, 0.0)
                for j in range(8):
                    s = pl.ds(16 * j, 16)
                    r = ab[q, k, s] + bb[q, k, s] + sp * vp[j] + sn * vn[j]
                    ab[q, k, s] = jnp.maximum(r, 0.0)

        def issue_scatter(q):
            pltpu.async_copy(ab.at[q], agg_sh.at[dstb.at[q]], ssem.at[q], add=True)

        def wait_scatter(q):
            pltpu.make_async_copy(ab.at[q], agg_sh.at[dstb.at[q]], ssem.at[q]).wait()

        issue_idx(0, 0)
        issue_idx(1, 1)
        wait_idx(0, 0)
        issue_gather(0)

        @pl.loop(0, chunks_per_tec // _NBUF)
        def _(g):
            for qq in range(_NBUF):
                t = g * _NBUF + qq
                nq = (qq + 1) % _NBUF
                pq = (qq + 2) % _NBUF

                @pl.when(t + 1 < chunks_per_tec)
                def _():
                    wait_idx(t + 1, nq)
                    issue_gather(nq)

                wait_gather(qq)
                compute(qq)
                issue_scatter(qq)

                @pl.when(t >= 1)
                def _():
                    wait_scatter(pq)

                @pl.when(t + 2 < chunks_per_tec)
                def _():
                    issue_idx(t + 2, pq)

        wait_scatter((chunks_per_tec - 1) % _NBUF)
        plsc.subcore_barrier()
        pltpu.sync_copy(agg_sh.at[pl.ds(row0, _ROWS_PER_TEC)],
                        out_hbm.at[c, pl.ds(row0, _ROWS_PER_TEC)])

    return sck(A, B, vv_flat, src, dst, ev)


def _tc_post(parts, ne, Wun, Wua, Wdn, Wdl, Wd2):
    n = ne.shape[0]
    blk = 2000
    grid = n // blk

    def body(p_ref, ne_ref, Wun_ref, Wua_ref, Wdn_ref, Wdl_ref, Wd2_ref, o_ref):
        agg = p_ref[0] + p_ref[1]
        ne = ne_ref[...]
        latent = jnp.maximum(
            jnp.dot(ne, Wun_ref[...], preferred_element_type=jnp.float32)
            + jnp.dot(agg, Wua_ref[...], preferred_element_type=jnp.float32), 0.0)
        dech = jnp.maximum(
            jnp.dot(ne, Wdn_ref[...], preferred_element_type=jnp.float32)
            + jnp.dot(latent, Wdl_ref[...], preferred_element_type=jnp.float32), 0.0)
        o_ref[...] = jnp.dot(dech, Wd2_ref[...], preferred_element_type=jnp.float32)

    return pl.pallas_call(
        body,
        grid=(grid,),
        in_specs=[
            pl.BlockSpec((2, blk, _L), lambda i: (0, i, 0)),
            pl.BlockSpec((blk, _L), lambda i: (i, 0)),
            pl.BlockSpec((_L, _L), lambda i: (0, 0)),
            pl.BlockSpec((_L, _L), lambda i: (0, 0)),
            pl.BlockSpec((_L, _L), lambda i: (0, 0)),
            pl.BlockSpec((_L, _L), lambda i: (0, 0)),
            pl.BlockSpec((_L, 1), lambda i: (0, 0)),
        ],
        out_specs=[pl.BlockSpec((blk, 1), lambda i: (i, 0))],
        out_shape=[jax.ShapeDtypeStruct((n, 1), jnp.float32)],
    )(parts, ne, Wun, Wua, Wdn, Wdl, Wd2)[0]


def kernel(node_features, edge_features, latent_features, edge_index,
           W_node_enc, W_edge_enc, W_msg, W_upd, W_dec1, W_dec2):
    n = node_features.shape[0]
    e_cnt = edge_features.shape[0]

    x = node_features.astype(jnp.float32)[:, None]
    h0 = latent_features.astype(jnp.float32)
    ev = edge_features.astype(jnp.float32)
    src = edge_index[0].astype(jnp.int32)
    dst = edge_index[1].astype(jnp.int32)

    epad = _ceil_to(e_cnt, _NC * _NS * _CHUNK * _NBUF)
    chunks_per_tec = epad // (_NC * _NS * _CHUNK)
    npad = epad - e_cnt
    src = jnp.concatenate([src, jnp.zeros((npad,), jnp.int32)])
    dst = jnp.concatenate([dst, jnp.full((npad,), _DUMMY, jnp.int32)])
    ev = jnp.concatenate([ev, jnp.zeros((npad,), jnp.float32)])

    w0 = W_node_enc[0:1]
    Wh = W_node_enc[1:]
    W1 = W_msg[0:_L]
    W2 = W_msg[_L:2 * _L]
    W3 = W_msg[2 * _L:]

    ne, A, B, vv = _tc_pre(x, h0, w0, Wh, W1, W2, W_edge_enc, W3)
    parts = _sc_edge(A, B, vv.reshape(-1), src, dst, ev, chunks_per_tec)
    out = _tc_post(parts, ne, W_upd[:_L], W_upd[_L:], W_dec1[:_L], W_dec1[_L:], W_dec2)
    return out
